# Initial kernel scaffold; baseline (speedup 1.0000x reference)
import functools

import jax
import jax.numpy as jnp
from jax import lax
from jax.experimental import pallas as pl
from jax.experimental.pallas import tpu as pltpu
from jax.experimental.pallas import tpu_sc as plsc

_N = 10000
_E = 320000
_D = 128
_DE = 16
_H = 128
_Q = 16
_G = 64
_STEPS = 2

_NC = 2
_NS = 16
_NW = _NC * _NS
_EPW = _E // _NW
_CH = 400
_NCH = _EPW // _CH
_RPT = _N // _NS

_CE = 2000
_GE = _E // _CE


def _sc_mesh():
    return plsc.VectorSubcoreMesh(core_axis_name="c", subcore_axis_name="s")


def _sc_gather(table, idx):
    @functools.partial(
        pl.kernel,
        out_type=jax.ShapeDtypeStruct((_E, _H), jnp.float32),
        mesh=_sc_mesh(),
        scratch_types=[
            pltpu.VMEM((_CH,), jnp.int32),
            pltpu.VMEM((_CH, _H), jnp.float32),
            pltpu.SemaphoreType.DMA,
        ],
    )
    def k(table_hbm, idx_hbm, out_hbm, idx_v, rows_v, sem):
        wid = lax.axis_index("s") * _NC + lax.axis_index("c")
        base = wid * _EPW

        def body(i, carry):
            off = base + i * _CH
            pltpu.sync_copy(idx_hbm.at[pl.ds(off, _CH)], idx_v)
            pltpu.async_copy(table_hbm.at[idx_v], rows_v, sem).wait()
            pltpu.sync_copy(rows_v, out_hbm.at[pl.ds(off, _CH)])
            return carry

        lax.fori_loop(0, _NCH, body, 0)

    return k(table, idx)


def _sc_segsum(values, idx, zeros):
    @functools.partial(
        pl.kernel,
        out_type=jax.ShapeDtypeStruct((_NC, _N, _H), jnp.float32),
        mesh=_sc_mesh(),
        scratch_types=[
            pltpu.VMEM((_CH,), jnp.int32),
            pltpu.VMEM((_CH, _H), jnp.float32),
            pltpu.VMEM_SHARED((_N, _H), jnp.float32),
        ],
    )
    def k(val_hbm, idx_hbm, zero_hbm, out_hbm, idx_v, val_v, acc):
        c = lax.axis_index("c")
        s = lax.axis_index("s")
        r0 = s * _RPT
        pltpu.sync_copy(zero_hbm.at[pl.ds(r0, _RPT)], acc.at[pl.ds(r0, _RPT)])
        plsc.subcore_barrier()

        wid = c * _NS + s
        base = wid * _EPW

        def body(i, carry):
            off = base + i * _CH
            pltpu.sync_copy(val_hbm.at[pl.ds(off, _CH)], val_v)
            pltpu.sync_copy(idx_hbm.at[pl.ds(off, _CH)], idx_v)
            pltpu.sync_copy(val_v, acc.at[idx_v], add=True)
            return carry

        lax.fori_loop(0, _NCH, body, 0)
        plsc.subcore_barrier()
        pltpu.sync_copy(acc.at[pl.ds(r0, _RPT)], out_hbm.at[c, pl.ds(r0, _RPT)])

    return k(values, idx, zeros)


def _full(shape):
    return pl.BlockSpec(shape, lambda *_: tuple(0 for _ in shape))


def _tc_edge_pre(edge_attr, We):
    def body(ea_ref, w_ref, out_ref):
        out_ref[...] = jnp.dot(ea_ref[...], w_ref[...],
                               preferred_element_type=jnp.float32)

    return pl.pallas_call(
        body,
        grid=(_GE,),
        in_specs=[
            pl.BlockSpec((_CE, _DE), lambda i: (i, 0)),
            pl.BlockSpec((_DE, _H), lambda i: (0, 0)),
        ],
        out_specs=pl.BlockSpec((_CE, _H), lambda i: (i, 0)),
        out_shape=jax.ShapeDtypeStruct((_E, _H), jnp.float32),
    )(edge_attr, We)


def _tc_node_pre(x, Wx, b):
    def body(x_ref, w_ref, b_ref, out_ref):
        out_ref[...] = jnp.dot(x_ref[...], w_ref[...],
                               preferred_element_type=jnp.float32) + b_ref[...]

    return pl.pallas_call(
        body,
        in_specs=[_full((_N, _D)), _full((_D, _H)), _full((1, _H))],
        out_specs=_full((_N, _H)),
        out_shape=jax.ShapeDtypeStruct((_N, _H), jnp.float32),
    )(x, Wx, b)


def _tc_init_combine(gxa, ea, Wh1):
    def body(g_ref, e_ref, w_ref, h0_ref, b0_ref):
        h0 = jnp.maximum(g_ref[...] + e_ref[...], 0.0)
        h0_ref[...] = h0
        b0_ref[...] = jnp.dot(h0, w_ref[...], preferred_element_type=jnp.float32)

    return pl.pallas_call(
        body,
        grid=(_GE,),
        in_specs=[
            pl.BlockSpec((_CE, _H), lambda i: (i, 0)),
            pl.BlockSpec((_CE, _H), lambda i: (i, 0)),
            pl.BlockSpec((_H, _H), lambda i: (0, 0)),
        ],
        out_specs=[
            pl.BlockSpec((_CE, _H), lambda i: (i, 0)),
            pl.BlockSpec((_CE, _H), lambda i: (i, 0)),
        ],
        out_shape=[
            jax.ShapeDtypeStruct((_E, _H), jnp.float32),
            jax.ShapeDtypeStruct((_E, _H), jnp.float32),
        ],
    )(gxa, ea, Wh1)


def _tc_a_table(s_part, Wh1, bh1):
    def body(s_ref, w_ref, b_ref, out_ref):
        ssum = s_ref[0] + s_ref[1]
        out_ref[...] = jnp.dot(ssum, w_ref[...],
                               preferred_element_type=jnp.float32) + b_ref[...]

    return pl.pallas_call(
        body,
        in_specs=[_full((_NC, _N, _H)), _full((_H, _H)), _full((1, _H))],
        out_specs=_full((_N, _H)),
        out_shape=jax.ShapeDtypeStruct((_N, _H), jnp.float32),
    )(s_part, Wh1, bh1)


def _tc_step_combine(ga, b_arr, h0, Wh2, bh2, Wh1, with_next_b):
    def body_b(ga_ref, b_ref, h0_ref, w2_ref, b2_ref, w1_ref, h_ref, bn_ref):
        u = jnp.maximum(ga_ref[...] - b_ref[...], 0.0)
        hid = jnp.dot(u, w2_ref[...], preferred_element_type=jnp.float32) + b2_ref[...]
        h = jnp.maximum(h0_ref[...] + hid, 0.0)
        h_ref[...] = h
        bn_ref[...] = jnp.dot(h, w1_ref[...], preferred_element_type=jnp.float32)

    def body_nb(ga_ref, b_ref, h0_ref, w2_ref, b2_ref, w1_ref, h_ref):
        u = jnp.maximum(ga_ref[...] - b_ref[...], 0.0)
        hid = jnp.dot(u, w2_ref[...], preferred_element_type=jnp.float32) + b2_ref[...]
        h_ref[...] = jnp.maximum(h0_ref[...] + hid, 0.0)

    blk = pl.BlockSpec((_CE, _H), lambda i: (i, 0))
    wblk = pl.BlockSpec((_H, _H), lambda i: (0, 0))
    bblk = pl.BlockSpec((1, _H), lambda i: (0, 0))
    eshape = jax.ShapeDtypeStruct((_E, _H), jnp.float32)
    if with_next_b:
        return pl.pallas_call(
            body_b, grid=(_GE,),
            in_specs=[blk, blk, blk, wblk, bblk, wblk],
            out_specs=[blk, blk],
            out_shape=[eshape, eshape],
        )(ga, b_arr, h0, Wh2, bh2, Wh1)
    out = pl.pallas_call(
        body_nb, grid=(_GE,),
        in_specs=[blk, blk, blk, wblk, bblk, wblk],
        out_specs=blk,
        out_shape=eshape,
    )(ga, b_arr, h0, Wh2, bh2, Wh1)
    return out, None


def _tc_final(x, m_part, batch2d, Wf1, Wf2, bf, gamma, beta, Wq):
    def body(x_ref, m_ref, bat_ref, wf1_ref, wf2_ref, bf_ref, g_ref, be_ref,
             wq_ref, out_ref):
        m_node = m_ref[0] + m_ref[1]
        h = jnp.dot(x_ref[...], wf1_ref[...], preferred_element_type=jnp.float32)
        h += jnp.dot(m_node, wf2_ref[...], preferred_element_type=jnp.float32)
        h = jnp.maximum(h + bf_ref[...], 0.0)
        mean = jnp.mean(h, axis=-1, keepdims=True)
        var = jnp.mean((h - mean) ** 2, axis=-1, keepdims=True)
        hn = (h - mean) * lax.rsqrt(var + 1e-5) * g_ref[...] + be_ref[...]
        hn = jnp.maximum(hn, 0.0)
        onehot = (bat_ref[...] ==
                  lax.broadcasted_iota(jnp.int32, (_N, _G), 1)).astype(jnp.float32)
        psum = lax.dot_general(onehot, hn, (((0,), (0,)), ((), ())),
                               preferred_element_type=jnp.float32)
        counts = lax.dot_general(onehot, jnp.ones((_N, 1), jnp.float32),
                                 (((0,), (0,)), ((), ())),
                                 preferred_element_type=jnp.float32)
        pooled = psum / jnp.maximum(counts, 1.0)
        out_ref[...] = jnp.dot(pooled, wq_ref[...],
                               preferred_element_type=jnp.float32)

    return pl.pallas_call(
        body,
        in_specs=[
            _full((_N, _D)), _full((_NC, _N, _H)), _full((_N, 1)),
            _full((_D, _H)), _full((_H, _H)), _full((1, _H)),
            _full((1, _H)), _full((1, _H)), _full((_H, _Q)),
        ],
        out_specs=_full((_G, _Q)),
        out_shape=jax.ShapeDtypeStruct((_G, _Q), jnp.float32),
    )(x, m_part, batch2d, Wf1, Wf2, bf, gamma, beta, Wq)


def kernel(x, edge_attr, edge_index, batch, W_init, b_init, Wh1, bh1, Wh2, bh2,
           W_fin, b_fin, gamma, beta, Wq):
    src = edge_index[0]
    dst = edge_index[1]
    Wx = W_init[:_D]
    We = W_init[_D:]
    Wf1 = W_fin[:_D]
    Wf2 = W_fin[_D:]
    b_init2 = b_init.reshape(1, _H)
    bh1_2 = bh1.reshape(1, _H)
    bh2_2 = bh2.reshape(1, _H)
    bf2 = b_fin.reshape(1, _H)
    gamma2 = gamma.reshape(1, _H)
    beta2 = beta.reshape(1, _H)
    zeros_nh = jnp.zeros((_N, _H), jnp.float32)

    XA = _tc_node_pre(x, Wx, b_init2)
    EA = _tc_edge_pre(edge_attr, We)
    gXA = _sc_gather(XA, src)
    h0, B = _tc_init_combine(gXA, EA, Wh1)

    h = h0
    for step in range(_STEPS):
        s_part = _sc_segsum(h, dst, zeros_nh)
        A = _tc_a_table(s_part, Wh1, bh1_2)
        gA = _sc_gather(A, dst)
        h, B = _tc_step_combine(gA, B, h0, Wh2, bh2_2, Wh1,
                                with_next_b=(step < _STEPS - 1))

    m_part = _sc_segsum(h, src, zeros_nh)
    return _tc_final(x, m_part, batch.reshape(_N, 1), Wf1, Wf2, bf2,
                     gamma2, beta2, Wq)


# speedup vs baseline: 2.0553x; 2.0553x over previous
import functools

import jax
import jax.numpy as jnp
from jax import lax
from jax.experimental import pallas as pl
from jax.experimental.pallas import tpu as pltpu
from jax.experimental.pallas import tpu_sc as plsc

_N = 10000
_E = 320000
_D = 128
_DE = 16
_H = 128
_Q = 16
_G = 64
_STEPS = 2

_NC = 2
_NS = 16
_NW = _NC * _NS
_EPW = _E // _NW
_GB = 80
_GSUB = 5
_CH = _GB * _GSUB
_NCH = _EPW // _CH
_SB = 40
_SSUB = 5
_CHS = _SB * _SSUB
_NCHS = _EPW // _CHS
_RPT = 624
_RLAST = _N - (_NS - 1) * _RPT

_CE = 2000
_GE = _E // _CE


def _sc_mesh():
    return plsc.VectorSubcoreMesh(core_axis_name="c", subcore_axis_name="s")


def _sc_gather(table, idx):
    @functools.partial(
        pl.kernel,
        out_type=jax.ShapeDtypeStruct((_E, _H), jnp.float32),
        mesh=_sc_mesh(),
        scratch_types=[
            pltpu.VMEM((_GSUB, _GB), jnp.int32),
            pltpu.VMEM((_CH, _H), jnp.float32),
            pltpu.SemaphoreType.DMA,
        ],
    )
    def k(table_hbm, idx_hbm, out_hbm, idx_v, rows_v, sem):
        wid = lax.axis_index("s") * _NC + lax.axis_index("c")
        base = wid * _EPW

        def body(i, carry):
            off = pl.multiple_of(base + i * _CH, 8)
            for j in range(_GSUB):
                pltpu.sync_copy(idx_hbm.at[pl.ds(off + j * _GB, _GB)],
                                idx_v.at[j])
            for j in range(_GSUB):
                pltpu.async_copy(table_hbm.at[idx_v.at[j]],
                                 rows_v.at[pl.ds(j * _GB, _GB)], sem).wait()
            pltpu.sync_copy(rows_v, out_hbm.at[pl.ds(off, _CH)])
            return carry

        lax.fori_loop(0, _NCH, body, 0)

    return k(table, idx)


def _sc_segsum(values, idx, zeros):
    @functools.partial(
        pl.kernel,
        out_type=jax.ShapeDtypeStruct((_NC, _N, _H), jnp.float32),
        mesh=_sc_mesh(),
        scratch_types=[
            pltpu.VMEM((_SSUB, _SB), jnp.int32),
            pltpu.VMEM((_CHS, _H), jnp.float32),
            pltpu.VMEM_SHARED((_N, _H), jnp.float32),
        ],
    )
    def k(val_hbm, idx_hbm, zero_hbm, out_hbm, idx_v, val_v, acc):
        c = lax.axis_index("c")
        s = lax.axis_index("s")
        r0 = pl.multiple_of(s * _RPT, 8)
        rlo = (_NS - 1) * _RPT

        @pl.when(s < _NS - 1)
        def _zero_main():
            pltpu.sync_copy(zero_hbm.at[pl.ds(r0, _RPT)],
                            acc.at[pl.ds(r0, _RPT)])

        @pl.when(s == _NS - 1)
        def _zero_last():
            pltpu.sync_copy(zero_hbm.at[pl.ds(rlo, _RLAST)],
                            acc.at[pl.ds(rlo, _RLAST)])

        plsc.subcore_barrier()

        wid = c * _NS + s
        base = wid * _EPW

        def body(i, carry):
            off = pl.multiple_of(base + i * _CHS, 8)
            pltpu.sync_copy(val_hbm.at[pl.ds(off, _CHS)], val_v)
            for j in range(_SSUB):
                pltpu.sync_copy(idx_hbm.at[pl.ds(off + j * _SB, _SB)],
                                idx_v.at[j])
            for j in range(_SSUB):
                pltpu.sync_copy(val_v.at[pl.ds(j * _SB, _SB)],
                                acc.at[idx_v.at[j]], add=True)
            return carry

        lax.fori_loop(0, _NCHS, body, 0)
        plsc.subcore_barrier()

        @pl.when(s < _NS - 1)
        def _out_main():
            pltpu.sync_copy(acc.at[pl.ds(r0, _RPT)],
                            out_hbm.at[c, pl.ds(r0, _RPT)])

        @pl.when(s == _NS - 1)
        def _out_last():
            pltpu.sync_copy(acc.at[pl.ds(rlo, _RLAST)],
                            out_hbm.at[c, pl.ds(rlo, _RLAST)])

    return k(values, idx, zeros)


def _full(shape):
    return pl.BlockSpec(shape, lambda *_: tuple(0 for _ in shape))


def _tc_edge_pre(edge_attr, We):
    def body(ea_ref, w_ref, out_ref):
        out_ref[...] = jnp.dot(ea_ref[...], w_ref[...],
                               preferred_element_type=jnp.float32)

    return pl.pallas_call(
        body,
        grid=(_GE,),
        in_specs=[
            pl.BlockSpec((_CE, _DE), lambda i: (i, 0)),
            pl.BlockSpec((_DE, _H), lambda i: (0, 0)),
        ],
        out_specs=pl.BlockSpec((_CE, _H), lambda i: (i, 0)),
        out_shape=jax.ShapeDtypeStruct((_E, _H), jnp.float32),
    )(edge_attr, We)


def _tc_node_pre(x, Wx, b):
    def body(x_ref, w_ref, b_ref, out_ref):
        out_ref[...] = jnp.dot(x_ref[...], w_ref[...],
                               preferred_element_type=jnp.float32) + b_ref[...]

    return pl.pallas_call(
        body,
        in_specs=[_full((_N, _D)), _full((_D, _H)), _full((1, _H))],
        out_specs=_full((_N, _H)),
        out_shape=jax.ShapeDtypeStruct((_N, _H), jnp.float32),
    )(x, Wx, b)


def _tc_init_combine(gxa, ea, Wh1):
    def body(g_ref, e_ref, w_ref, h0_ref, b0_ref):
        h0 = jnp.maximum(g_ref[...] + e_ref[...], 0.0)
        h0_ref[...] = h0
        b0_ref[...] = jnp.dot(h0, w_ref[...], preferred_element_type=jnp.float32)

    return pl.pallas_call(
        body,
        grid=(_GE,),
        in_specs=[
            pl.BlockSpec((_CE, _H), lambda i: (i, 0)),
            pl.BlockSpec((_CE, _H), lambda i: (i, 0)),
            pl.BlockSpec((_H, _H), lambda i: (0, 0)),
        ],
        out_specs=[
            pl.BlockSpec((_CE, _H), lambda i: (i, 0)),
            pl.BlockSpec((_CE, _H), lambda i: (i, 0)),
        ],
        out_shape=[
            jax.ShapeDtypeStruct((_E, _H), jnp.float32),
            jax.ShapeDtypeStruct((_E, _H), jnp.float32),
        ],
    )(gxa, ea, Wh1)


def _tc_a_table(s_part, Wh1, bh1):
    def body(s_ref, w_ref, b_ref, out_ref):
        ssum = s_ref[0] + s_ref[1]
        out_ref[...] = jnp.dot(ssum, w_ref[...],
                               preferred_element_type=jnp.float32) + b_ref[...]

    return pl.pallas_call(
        body,
        in_specs=[_full((_NC, _N, _H)), _full((_H, _H)), _full((1, _H))],
        out_specs=_full((_N, _H)),
        out_shape=jax.ShapeDtypeStruct((_N, _H), jnp.float32),
    )(s_part, Wh1, bh1)


def _tc_step_combine(ga, b_arr, h0, Wh2, bh2, Wh1, with_next_b):
    def body_b(ga_ref, b_ref, h0_ref, w2_ref, b2_ref, w1_ref, h_ref, bn_ref):
        u = jnp.maximum(ga_ref[...] - b_ref[...], 0.0)
        hid = jnp.dot(u, w2_ref[...], preferred_element_type=jnp.float32) + b2_ref[...]
        h = jnp.maximum(h0_ref[...] + hid, 0.0)
        h_ref[...] = h
        bn_ref[...] = jnp.dot(h, w1_ref[...], preferred_element_type=jnp.float32)

    def body_nb(ga_ref, b_ref, h0_ref, w2_ref, b2_ref, w1_ref, h_ref):
        u = jnp.maximum(ga_ref[...] - b_ref[...], 0.0)
        hid = jnp.dot(u, w2_ref[...], preferred_element_type=jnp.float32) + b2_ref[...]
        h_ref[...] = jnp.maximum(h0_ref[...] + hid, 0.0)

    blk = pl.BlockSpec((_CE, _H), lambda i: (i, 0))
    wblk = pl.BlockSpec((_H, _H), lambda i: (0, 0))
    bblk = pl.BlockSpec((1, _H), lambda i: (0, 0))
    eshape = jax.ShapeDtypeStruct((_E, _H), jnp.float32)
    if with_next_b:
        return pl.pallas_call(
            body_b, grid=(_GE,),
            in_specs=[blk, blk, blk, wblk, bblk, wblk],
            out_specs=[blk, blk],
            out_shape=[eshape, eshape],
        )(ga, b_arr, h0, Wh2, bh2, Wh1)
    out = pl.pallas_call(
        body_nb, grid=(_GE,),
        in_specs=[blk, blk, blk, wblk, bblk, wblk],
        out_specs=blk,
        out_shape=eshape,
    )(ga, b_arr, h0, Wh2, bh2, Wh1)
    return out, None


def _tc_final(x, m_part, batch2d, Wf1, Wf2, bf, gamma, beta, Wq):
    def body(x_ref, m_ref, bat_ref, wf1_ref, wf2_ref, bf_ref, g_ref, be_ref,
             wq_ref, out_ref):
        m_node = m_ref[0] + m_ref[1]
        h = jnp.dot(x_ref[...], wf1_ref[...], preferred_element_type=jnp.float32)
        h += jnp.dot(m_node, wf2_ref[...], preferred_element_type=jnp.float32)
        h = jnp.maximum(h + bf_ref[...], 0.0)
        mean = jnp.mean(h, axis=-1, keepdims=True)
        var = jnp.mean((h - mean) ** 2, axis=-1, keepdims=True)
        hn = (h - mean) * lax.rsqrt(var + 1e-5) * g_ref[...] + be_ref[...]
        hn = jnp.maximum(hn, 0.0)
        onehot = (bat_ref[...] ==
                  lax.broadcasted_iota(jnp.int32, (_N, _G), 1)).astype(jnp.float32)
        psum = lax.dot_general(onehot, hn, (((0,), (0,)), ((), ())),
                               preferred_element_type=jnp.float32)
        counts = lax.dot_general(onehot, jnp.ones((_N, 1), jnp.float32),
                                 (((0,), (0,)), ((), ())),
                                 preferred_element_type=jnp.float32)
        pooled = psum / jnp.maximum(counts, 1.0)
        out_ref[...] = jnp.dot(pooled, wq_ref[...],
                               preferred_element_type=jnp.float32)

    return pl.pallas_call(
        body,
        in_specs=[
            _full((_N, _D)), _full((_NC, _N, _H)), _full((_N, 1)),
            _full((_D, _H)), _full((_H, _H)), _full((1, _H)),
            _full((1, _H)), _full((1, _H)), _full((_H, _Q)),
        ],
        out_specs=_full((_G, _Q)),
        out_shape=jax.ShapeDtypeStruct((_G, _Q), jnp.float32),
    )(x, m_part, batch2d, Wf1, Wf2, bf, gamma, beta, Wq)


def kernel(x, edge_attr, edge_index, batch, W_init, b_init, Wh1, bh1, Wh2, bh2,
           W_fin, b_fin, gamma, beta, Wq):
    src = edge_index[0]
    dst = edge_index[1]
    Wx = W_init[:_D]
    We = W_init[_D:]
    Wf1 = W_fin[:_D]
    Wf2 = W_fin[_D:]
    b_init2 = b_init.reshape(1, _H)
    bh1_2 = bh1.reshape(1, _H)
    bh2_2 = bh2.reshape(1, _H)
    bf2 = b_fin.reshape(1, _H)
    gamma2 = gamma.reshape(1, _H)
    beta2 = beta.reshape(1, _H)
    zeros_nh = jnp.zeros((_N, _H), jnp.float32)

    XA = _tc_node_pre(x, Wx, b_init2)
    EA = _tc_edge_pre(edge_attr, We)
    gXA = _sc_gather(XA, src)
    h0, B = _tc_init_combine(gXA, EA, Wh1)

    h = h0
    for step in range(_STEPS):
        s_part = _sc_segsum(h, dst, zeros_nh)
        A = _tc_a_table(s_part, Wh1, bh1_2)
        gA = _sc_gather(A, dst)
        h, B = _tc_step_combine(gA, B, h0, Wh2, bh2_2, Wh1,
                                with_next_b=(step < _STEPS - 1))

    m_part = _sc_segsum(h, src, zeros_nh)
    return _tc_final(x, m_part, batch.reshape(_N, 1), Wf1, Wf2, bf2,
                     gamma2, beta2, Wq)


# speedup vs baseline: 2.1157x; 1.0294x over previous
import functools

import jax
import jax.numpy as jnp
from jax import lax
from jax.experimental import pallas as pl
from jax.experimental.pallas import tpu as pltpu
from jax.experimental.pallas import tpu_sc as plsc

_N = 10000
_E = 320000
_D = 128
_DE = 16
_H = 128
_Q = 16
_G = 64
_STEPS = 2

_NC = 2
_NS = 16
_NW = _NC * _NS
_EPW = _E // _NW
_GB = 80
_GSUB = 5
_CH = _GB * _GSUB
_NCH = _EPW // _CH
_SB = 40
_SSUB = 5
_CHS = _SB * _SSUB
_NCHS = _EPW // _CHS
_RPT = 624
_RLAST = _N - (_NS - 1) * _RPT

_CE = 2000
_GE = _E // _CE


def _sc_mesh():
    return plsc.VectorSubcoreMesh(core_axis_name="c", subcore_axis_name="s")


def _sc_gather(table, idx):
    @functools.partial(
        pl.kernel,
        out_type=jax.ShapeDtypeStruct((_E, _H), jnp.float32),
        mesh=_sc_mesh(),
        scratch_types=[
            pltpu.VMEM((_GSUB, _GB), jnp.int32),
            pltpu.VMEM((_CH, _H), jnp.float32),
            pltpu.SemaphoreType.DMA,
        ],
    )
    def k(table_hbm, idx_hbm, out_hbm, idx_v, rows_v, sem):
        wid = lax.axis_index("s") * _NC + lax.axis_index("c")
        base = wid * _EPW

        def body(i, carry):
            off = pl.multiple_of(base + i * _CH, 8)
            for j in range(_GSUB):
                pltpu.sync_copy(idx_hbm.at[pl.ds(off + j * _GB, _GB)],
                                idx_v.at[j])
            for j in range(_GSUB):
                pltpu.async_copy(table_hbm.at[idx_v.at[j]],
                                 rows_v.at[pl.ds(j * _GB, _GB)], sem).wait()
            pltpu.sync_copy(rows_v, out_hbm.at[pl.ds(off, _CH)])
            return carry

        lax.fori_loop(0, _NCH, body, 0)

    return k(table, idx)


def _sc_segsum(values, idx, zeros):
    @functools.partial(
        pl.kernel,
        out_type=jax.ShapeDtypeStruct((_NC, _N, _H), jnp.float32),
        mesh=_sc_mesh(),
        scratch_types=[
            pltpu.VMEM((_SSUB, _SB), jnp.int32),
            pltpu.VMEM((_CHS, _H), jnp.float32),
            pltpu.VMEM_SHARED((_N, _H), jnp.float32),
        ],
    )
    def k(val_hbm, idx_hbm, zero_hbm, out_hbm, idx_v, val_v, acc):
        c = lax.axis_index("c")
        s = lax.axis_index("s")
        r0 = pl.multiple_of(s * _RPT, 8)
        rlo = (_NS - 1) * _RPT

        @pl.when(s < _NS - 1)
        def _zero_main():
            pltpu.sync_copy(zero_hbm.at[pl.ds(r0, _RPT)],
                            acc.at[pl.ds(r0, _RPT)])

        @pl.when(s == _NS - 1)
        def _zero_last():
            pltpu.sync_copy(zero_hbm.at[pl.ds(rlo, _RLAST)],
                            acc.at[pl.ds(rlo, _RLAST)])

        plsc.subcore_barrier()

        wid = c * _NS + s
        base = wid * _EPW

        def body(i, carry):
            off = pl.multiple_of(base + i * _CHS, 8)
            pltpu.sync_copy(val_hbm.at[pl.ds(off, _CHS)], val_v)
            for j in range(_SSUB):
                pltpu.sync_copy(idx_hbm.at[pl.ds(off + j * _SB, _SB)],
                                idx_v.at[j])
            for j in range(_SSUB):
                pltpu.sync_copy(val_v.at[pl.ds(j * _SB, _SB)],
                                acc.at[idx_v.at[j]], add=True)
            return carry

        lax.fori_loop(0, _NCHS, body, 0)
        plsc.subcore_barrier()

        @pl.when(s < _NS - 1)
        def _out_main():
            pltpu.sync_copy(acc.at[pl.ds(r0, _RPT)],
                            out_hbm.at[c, pl.ds(r0, _RPT)])

        @pl.when(s == _NS - 1)
        def _out_last():
            pltpu.sync_copy(acc.at[pl.ds(rlo, _RLAST)],
                            out_hbm.at[c, pl.ds(rlo, _RLAST)])

    return k(values, idx, zeros)


def _full(shape):
    return pl.BlockSpec(shape, lambda *_: tuple(0 for _ in shape))


def _tc_edge_pre(edge_attr, We):
    def body(ea_ref, w_ref, out_ref):
        out_ref[...] = jnp.dot(ea_ref[...], w_ref[...],
                               preferred_element_type=jnp.float32)

    return pl.pallas_call(
        body,
        grid=(_GE,),
        in_specs=[
            pl.BlockSpec((_CE, _DE), lambda i: (i, 0)),
            pl.BlockSpec((_DE, _H), lambda i: (0, 0)),
        ],
        out_specs=pl.BlockSpec((_CE, _H), lambda i: (i, 0)),
        out_shape=jax.ShapeDtypeStruct((_E, _H), jnp.float32),
    )(edge_attr, We)


def _tc_node_pre(x, Wx, b):
    def body(x_ref, w_ref, b_ref, out_ref):
        out_ref[...] = jnp.dot(x_ref[...], w_ref[...],
                               preferred_element_type=jnp.float32) + b_ref[...]

    return pl.pallas_call(
        body,
        in_specs=[_full((_N, _D)), _full((_D, _H)), _full((1, _H))],
        out_specs=_full((_N, _H)),
        out_shape=jax.ShapeDtypeStruct((_N, _H), jnp.float32),
    )(x, Wx, b)


def _tc_init_combine(gxa, ea):
    def body(g_ref, e_ref, h0_ref):
        h0_ref[...] = jnp.maximum(g_ref[...] + e_ref[...], 0.0)

    blk = pl.BlockSpec((_CE, _H), lambda i: (i, 0))
    return pl.pallas_call(
        body,
        grid=(_GE,),
        in_specs=[blk, blk],
        out_specs=blk,
        out_shape=jax.ShapeDtypeStruct((_E, _H), jnp.float32),
    )(gxa, ea)


def _tc_combine(s_part):
    def body(s_ref, out_ref):
        out_ref[...] = s_ref[0] + s_ref[1]

    return pl.pallas_call(
        body,
        in_specs=[_full((_NC, _N, _H))],
        out_specs=_full((_N, _H)),
        out_shape=jax.ShapeDtypeStruct((_N, _H), jnp.float32),
    )(s_part)


def _tc_step(gs, h, h0, Wh1, bh1, Wh2, bh2):
    def body(gs_ref, h_ref, h0_ref, w1_ref, b1_ref, w2_ref, b2_ref, out_ref):
        m = gs_ref[...] - h_ref[...]
        u = jnp.maximum(
            jnp.dot(m, w1_ref[...], preferred_element_type=jnp.float32)
            + b1_ref[...], 0.0)
        hid = jnp.dot(u, w2_ref[...], preferred_element_type=jnp.float32)
        out_ref[...] = jnp.maximum(h0_ref[...] + hid + b2_ref[...], 0.0)

    blk = pl.BlockSpec((_CE, _H), lambda i: (i, 0))
    wblk = pl.BlockSpec((_H, _H), lambda i: (0, 0))
    bblk = pl.BlockSpec((1, _H), lambda i: (0, 0))
    return pl.pallas_call(
        body, grid=(_GE,),
        in_specs=[blk, blk, blk, wblk, bblk, wblk, bblk],
        out_specs=blk,
        out_shape=jax.ShapeDtypeStruct((_E, _H), jnp.float32),
    )(gs, h, h0, Wh1, bh1, Wh2, bh2)


def _tc_final(x, m_part, batch2d, Wf1, Wf2, bf, gamma, beta, Wq):
    def body(x_ref, m_ref, bat_ref, wf1_ref, wf2_ref, bf_ref, g_ref, be_ref,
             wq_ref, out_ref):
        m_node = m_ref[0] + m_ref[1]
        h = jnp.dot(x_ref[...], wf1_ref[...], preferred_element_type=jnp.float32)
        h += jnp.dot(m_node, wf2_ref[...], preferred_element_type=jnp.float32)
        h = jnp.maximum(h + bf_ref[...], 0.0)
        mean = jnp.mean(h, axis=-1, keepdims=True)
        var = jnp.mean((h - mean) ** 2, axis=-1, keepdims=True)
        hn = (h - mean) * lax.rsqrt(var + 1e-5) * g_ref[...] + be_ref[...]
        hn = jnp.maximum(hn, 0.0)
        onehot = (bat_ref[...] ==
                  lax.broadcasted_iota(jnp.int32, (_N, _G), 1)).astype(jnp.float32)
        psum = lax.dot_general(onehot, hn, (((0,), (0,)), ((), ())),
                               preferred_element_type=jnp.float32)
        counts = lax.dot_general(onehot, jnp.ones((_N, 1), jnp.float32),
                                 (((0,), (0,)), ((), ())),
                                 preferred_element_type=jnp.float32)
        pooled = psum / jnp.maximum(counts, 1.0)
        out_ref[...] = jnp.dot(pooled, wq_ref[...],
                               preferred_element_type=jnp.float32)

    return pl.pallas_call(
        body,
        in_specs=[
            _full((_N, _D)), _full((_NC, _N, _H)), _full((_N, 1)),
            _full((_D, _H)), _full((_H, _H)), _full((1, _H)),
            _full((1, _H)), _full((1, _H)), _full((_H, _Q)),
        ],
        out_specs=_full((_G, _Q)),
        out_shape=jax.ShapeDtypeStruct((_G, _Q), jnp.float32),
    )(x, m_part, batch2d, Wf1, Wf2, bf, gamma, beta, Wq)


def kernel(x, edge_attr, edge_index, batch, W_init, b_init, Wh1, bh1, Wh2, bh2,
           W_fin, b_fin, gamma, beta, Wq):
    src = edge_index[0]
    dst = edge_index[1]
    Wx = W_init[:_D]
    We = W_init[_D:]
    Wf1 = W_fin[:_D]
    Wf2 = W_fin[_D:]
    b_init2 = b_init.reshape(1, _H)
    bh1_2 = bh1.reshape(1, _H)
    bh2_2 = bh2.reshape(1, _H)
    bf2 = b_fin.reshape(1, _H)
    gamma2 = gamma.reshape(1, _H)
    beta2 = beta.reshape(1, _H)
    zeros_nh = jnp.zeros((_N, _H), jnp.float32)

    XA = _tc_node_pre(x, Wx, b_init2)
    EA = _tc_edge_pre(edge_attr, We)
    gXA = _sc_gather(XA, src)
    h0 = _tc_init_combine(gXA, EA)

    h = h0
    for _ in range(_STEPS):
        s_part = _sc_segsum(h, dst, zeros_nh)
        S = _tc_combine(s_part)
        gS = _sc_gather(S, dst)
        h = _tc_step(gS, h, h0, Wh1, bh1_2, Wh2, bh2_2)

    m_part = _sc_segsum(h, src, zeros_nh)
    return _tc_final(x, m_part, batch.reshape(_N, 1), Wf1, Wf2, bf2,
                     gamma2, beta2, Wq)


# speedup vs baseline: 2.8067x; 1.3266x over previous
import functools

import jax
import jax.numpy as jnp
from jax import lax
from jax.experimental import pallas as pl
from jax.experimental.pallas import tpu as pltpu
from jax.experimental.pallas import tpu_sc as plsc

_N = 10000
_E = 320000
_D = 128
_DE = 16
_H = 128
_Q = 16
_G = 64
_STEPS = 2

_NC = 2
_NS = 16
_NW = _NC * _NS
_EPW = _E // _NW
_GB = 80
_NSUB = _EPW // _GB
_GSUB = 5
_CH = _GB * _GSUB
_NCH = _EPW // _CH
_RPT = 624
_RLAST = _N - (_NS - 1) * _RPT

_CE = 2000
_GE = _E // _CE


def _sc_mesh():
    return plsc.VectorSubcoreMesh(core_axis_name="c", subcore_axis_name="s")


def _sc_gather(table, idx3d):
    @functools.partial(
        pl.kernel,
        out_type=jax.ShapeDtypeStruct((_E, _H), jnp.float32),
        mesh=_sc_mesh(),
        scratch_types=[
            pltpu.VMEM((_NSUB, _GB), jnp.int32),
            pltpu.VMEM((2, _CH, _H), jnp.float32),
            pltpu.SemaphoreType.DMA,
            pltpu.SemaphoreType.DMA,
            pltpu.SemaphoreType.DMA,
        ],
    )
    def k(table_hbm, idx_hbm, out_hbm, idx_v, rows_v, gsem, wsem0, wsem1):
        wid = lax.axis_index("s") * _NC + lax.axis_index("c")
        base = wid * _EPW
        pltpu.sync_copy(idx_hbm.at[wid], idx_v)

        def chunk(i, b, first):
            if not first:
                pltpu.make_async_copy(
                    rows_v.at[b], out_hbm.at[pl.ds(0, _CH)],
                    wsem0 if b == 0 else wsem1).wait()
            ds = [pltpu.async_copy(table_hbm.at[idx_v.at[i * _GSUB + j]],
                                   rows_v.at[b, pl.ds(j * _GB, _GB)], gsem)
                  for j in range(_GSUB)]
            for d in ds:
                d.wait()
            off = pl.multiple_of(base + i * _CH, 8)
            pltpu.async_copy(rows_v.at[b], out_hbm.at[pl.ds(off, _CH)],
                             wsem0 if b == 0 else wsem1)

        def prologue():
            chunk(0, 0, True)
            chunk(1, 1, True)

        prologue()

        def body(i2, carry):
            chunk(2 + i2 * 2, 0, False)
            chunk(3 + i2 * 2, 1, False)
            return carry

        lax.fori_loop(0, (_NCH - 3) // 2, body, 0)
        chunk(_NCH - 1, 0, False)
        pltpu.make_async_copy(rows_v.at[0], out_hbm.at[pl.ds(0, _CH)],
                              wsem0).wait()
        pltpu.make_async_copy(rows_v.at[1], out_hbm.at[pl.ds(0, _CH)],
                              wsem1).wait()

    return k(table, idx3d)


def _sc_segsum(values, idx3d, zeros):
    @functools.partial(
        pl.kernel,
        out_type=jax.ShapeDtypeStruct((_NC, _N, _H), jnp.float32),
        mesh=_sc_mesh(),
        scratch_types=[
            pltpu.VMEM((_NSUB, _GB), jnp.int32),
            pltpu.VMEM((2, _GB, _H), jnp.float32),
            pltpu.VMEM_SHARED((_N, _H), jnp.float32),
            pltpu.SemaphoreType.DMA,
            pltpu.SemaphoreType.DMA,
        ],
    )
    def k(val_hbm, idx_hbm, zero_hbm, out_hbm, idx_v, val_v, acc,
          lsem0, lsem1):
        c = lax.axis_index("c")
        s = lax.axis_index("s")
        r0 = pl.multiple_of(s * _RPT, 8)
        rlo = (_NS - 1) * _RPT

        @pl.when(s < _NS - 1)
        def _zero_main():
            pltpu.sync_copy(zero_hbm.at[pl.ds(r0, _RPT)],
                            acc.at[pl.ds(r0, _RPT)])

        @pl.when(s == _NS - 1)
        def _zero_last():
            pltpu.sync_copy(zero_hbm.at[pl.ds(rlo, _RLAST)],
                            acc.at[pl.ds(rlo, _RLAST)])

        plsc.subcore_barrier()

        wid = lax.axis_index("s") * _NC + lax.axis_index("c")
        base = wid * _EPW
        pltpu.sync_copy(idx_hbm.at[wid], idx_v)

        def load(i, b):
            off = pl.multiple_of(base + i * _GB, 8)
            pltpu.async_copy(val_hbm.at[pl.ds(off, _GB)], val_v.at[b],
                             lsem0 if b == 0 else lsem1)

        def drain(b):
            pltpu.make_async_copy(val_hbm.at[pl.ds(0, _GB)], val_v.at[b],
                                  lsem0 if b == 0 else lsem1).wait()

        def scat(i, b):
            pltpu.sync_copy(val_v.at[b], acc.at[idx_v.at[i]], add=True)

        load(0, 0)

        def body(i2, carry):
            i0 = i2 * 2
            drain(0)
            load(i0 + 1, 1)
            scat(i0, 0)
            drain(1)
            load(i0 + 2, 0)
            scat(i0 + 1, 1)
            return carry

        lax.fori_loop(0, (_NSUB - 1) // 2, body, 0)
        drain(0)
        scat(_NSUB - 1, 0)

        plsc.subcore_barrier()

        @pl.when(s < _NS - 1)
        def _out_main():
            pltpu.sync_copy(acc.at[pl.ds(r0, _RPT)],
                            out_hbm.at[c, pl.ds(r0, _RPT)])

        @pl.when(s == _NS - 1)
        def _out_last():
            pltpu.sync_copy(acc.at[pl.ds(rlo, _RLAST)],
                            out_hbm.at[c, pl.ds(rlo, _RLAST)])

    return k(values, idx3d, zeros)


def _full(shape):
    return pl.BlockSpec(shape, lambda *_: tuple(0 for _ in shape))


def _tc_edge_pre(edge_attr, We):
    def body(ea_ref, w_ref, out_ref):
        out_ref[...] = jnp.dot(ea_ref[...], w_ref[...],
                               preferred_element_type=jnp.float32)

    return pl.pallas_call(
        body,
        grid=(_GE,),
        in_specs=[
            pl.BlockSpec((_CE, _DE), lambda i: (i, 0)),
            pl.BlockSpec((_DE, _H), lambda i: (0, 0)),
        ],
        out_specs=pl.BlockSpec((_CE, _H), lambda i: (i, 0)),
        out_shape=jax.ShapeDtypeStruct((_E, _H), jnp.float32),
    )(edge_attr, We)


def _tc_node_pre(x, Wx, b):
    def body(x_ref, w_ref, b_ref, out_ref):
        out_ref[...] = jnp.dot(x_ref[...], w_ref[...],
                               preferred_element_type=jnp.float32) + b_ref[...]

    return pl.pallas_call(
        body,
        in_specs=[_full((_N, _D)), _full((_D, _H)), _full((1, _H))],
        out_specs=_full((_N, _H)),
        out_shape=jax.ShapeDtypeStruct((_N, _H), jnp.float32),
    )(x, Wx, b)


def _tc_init_combine(gxa, ea):
    def body(g_ref, e_ref, h0_ref):
        h0_ref[...] = jnp.maximum(g_ref[...] + e_ref[...], 0.0)

    blk = pl.BlockSpec((_CE, _H), lambda i: (i, 0))
    return pl.pallas_call(
        body,
        grid=(_GE,),
        in_specs=[blk, blk],
        out_specs=blk,
        out_shape=jax.ShapeDtypeStruct((_E, _H), jnp.float32),
    )(gxa, ea)


def _tc_combine(s_part):
    def body(s_ref, out_ref):
        out_ref[...] = s_ref[0] + s_ref[1]

    return pl.pallas_call(
        body,
        in_specs=[_full((_NC, _N, _H))],
        out_specs=_full((_N, _H)),
        out_shape=jax.ShapeDtypeStruct((_N, _H), jnp.float32),
    )(s_part)


def _tc_step(gs, h, h0, Wh1, bh1, Wh2, bh2):
    def body(gs_ref, h_ref, h0_ref, w1_ref, b1_ref, w2_ref, b2_ref, out_ref):
        m = gs_ref[...] - h_ref[...]
        u = jnp.maximum(
            jnp.dot(m, w1_ref[...], preferred_element_type=jnp.float32)
            + b1_ref[...], 0.0)
        hid = jnp.dot(u, w2_ref[...], preferred_element_type=jnp.float32)
        out_ref[...] = jnp.maximum(h0_ref[...] + hid + b2_ref[...], 0.0)

    blk = pl.BlockSpec((_CE, _H), lambda i: (i, 0))
    wblk = pl.BlockSpec((_H, _H), lambda i: (0, 0))
    bblk = pl.BlockSpec((1, _H), lambda i: (0, 0))
    return pl.pallas_call(
        body, grid=(_GE,),
        in_specs=[blk, blk, blk, wblk, bblk, wblk, bblk],
        out_specs=blk,
        out_shape=jax.ShapeDtypeStruct((_E, _H), jnp.float32),
    )(gs, h, h0, Wh1, bh1, Wh2, bh2)


def _tc_final(x, m_part, batch2d, Wf1, Wf2, bf, gamma, beta, Wq):
    def body(x_ref, m_ref, bat_ref, wf1_ref, wf2_ref, bf_ref, g_ref, be_ref,
             wq_ref, out_ref):
        m_node = m_ref[0] + m_ref[1]
        h = jnp.dot(x_ref[...], wf1_ref[...], preferred_element_type=jnp.float32)
        h += jnp.dot(m_node, wf2_ref[...], preferred_element_type=jnp.float32)
        h = jnp.maximum(h + bf_ref[...], 0.0)
        mean = jnp.mean(h, axis=-1, keepdims=True)
        var = jnp.mean((h - mean) ** 2, axis=-1, keepdims=True)
        hn = (h - mean) * lax.rsqrt(var + 1e-5) * g_ref[...] + be_ref[...]
        hn = jnp.maximum(hn, 0.0)
        onehot = (bat_ref[...] ==
                  lax.broadcasted_iota(jnp.int32, (_N, _G), 1)).astype(jnp.float32)
        psum = lax.dot_general(onehot, hn, (((0,), (0,)), ((), ())),
                               preferred_element_type=jnp.float32)
        counts = lax.dot_general(onehot, jnp.ones((_N, 1), jnp.float32),
                                 (((0,), (0,)), ((), ())),
                                 preferred_element_type=jnp.float32)
        pooled = psum / jnp.maximum(counts, 1.0)
        out_ref[...] = jnp.dot(pooled, wq_ref[...],
                               preferred_element_type=jnp.float32)

    return pl.pallas_call(
        body,
        in_specs=[
            _full((_N, _D)), _full((_NC, _N, _H)), _full((_N, 1)),
            _full((_D, _H)), _full((_H, _H)), _full((1, _H)),
            _full((1, _H)), _full((1, _H)), _full((_H, _Q)),
        ],
        out_specs=_full((_G, _Q)),
        out_shape=jax.ShapeDtypeStruct((_G, _Q), jnp.float32),
    )(x, m_part, batch2d, Wf1, Wf2, bf, gamma, beta, Wq)


def kernel(x, edge_attr, edge_index, batch, W_init, b_init, Wh1, bh1, Wh2, bh2,
           W_fin, b_fin, gamma, beta, Wq):
    src3d = edge_index[0].reshape(_NW, _NSUB, _GB)
    dst3d = edge_index[1].reshape(_NW, _NSUB, _GB)
    Wx = W_init[:_D]
    We = W_init[_D:]
    Wf1 = W_fin[:_D]
    Wf2 = W_fin[_D:]
    b_init2 = b_init.reshape(1, _H)
    bh1_2 = bh1.reshape(1, _H)
    bh2_2 = bh2.reshape(1, _H)
    bf2 = b_fin.reshape(1, _H)
    gamma2 = gamma.reshape(1, _H)
    beta2 = beta.reshape(1, _H)
    zeros_nh = jnp.zeros((_N, _H), jnp.float32)

    XA = _tc_node_pre(x, Wx, b_init2)
    EA = _tc_edge_pre(edge_attr, We)
    gXA = _sc_gather(XA, src3d)
    h0 = _tc_init_combine(gXA, EA)

    h = h0
    for _ in range(_STEPS):
        s_part = _sc_segsum(h, dst3d, zeros_nh)
        S = _tc_combine(s_part)
        gS = _sc_gather(S, dst3d)
        h = _tc_step(gS, h, h0, Wh1, bh1_2, Wh2, bh2_2)

    m_part = _sc_segsum(h, src3d, zeros_nh)
    return _tc_final(x, m_part, batch.reshape(_N, 1), Wf1, Wf2, bf2,
                     gamma2, beta2, Wq)


# speedup vs baseline: 2.9349x; 1.0457x over previous
import functools

import jax
import jax.numpy as jnp
from jax import lax
from jax.experimental import pallas as pl
from jax.experimental.pallas import tpu as pltpu
from jax.experimental.pallas import tpu_sc as plsc

_N = 10000
_E = 320000
_D = 128
_DE = 16
_H = 128
_Q = 16
_G = 64
_STEPS = 2

_NC = 2
_NS = 16
_NW = _NC * _NS
_NSLAB = 2
_ES = _E // _NSLAB
_EPW = _ES // _NW
_GB = 40
_NSUB = _EPW // _GB
_GSUB = 5
_CH = _GB * _GSUB
_NCH = _EPW // _CH
_RPT = 624
_RLAST = _N - (_NS - 1) * _RPT

_CE = 2000
_GE = _ES // _CE


def _sc_mesh():
    return plsc.VectorSubcoreMesh(core_axis_name="c", subcore_axis_name="s")


@functools.lru_cache(maxsize=None)
def _mk_sc_gather(slab):
    @functools.partial(
        pl.kernel,
        out_type=jax.ShapeDtypeStruct((_ES, _H), jnp.float32),
        mesh=_sc_mesh(),
        scratch_types=[
            pltpu.VMEM((_NSUB, _GB), jnp.int32),
            pltpu.VMEM((2, _CH, _H), jnp.float32),
            pltpu.SemaphoreType.DMA,
            pltpu.SemaphoreType.DMA,
            pltpu.SemaphoreType.DMA,
        ],
    )
    def k(table_hbm, idx_hbm, out_hbm, idx_v, rows_v, gsem, wsem0, wsem1):
        wid = lax.axis_index("s") * _NC + lax.axis_index("c")
        base = wid * _EPW
        pltpu.sync_copy(idx_hbm.at[slab, wid], idx_v)

        def chunk(i, b, first):
            if not first:
                pltpu.make_async_copy(
                    rows_v.at[b], out_hbm.at[pl.ds(0, _CH)],
                    wsem0 if b == 0 else wsem1).wait()
            ds = [pltpu.async_copy(table_hbm.at[idx_v.at[i * _GSUB + j]],
                                   rows_v.at[b, pl.ds(j * _GB, _GB)], gsem)
                  for j in range(_GSUB)]
            for d in ds:
                d.wait()
            off = pl.multiple_of(base + i * _CH, 8)
            pltpu.async_copy(rows_v.at[b], out_hbm.at[pl.ds(off, _CH)],
                             wsem0 if b == 0 else wsem1)

        chunk(0, 0, True)
        chunk(1, 1, True)

        def body(i2, carry):
            chunk(2 + i2 * 2, 0, False)
            chunk(3 + i2 * 2, 1, False)
            return carry

        lax.fori_loop(0, (_NCH - 3) // 2, body, 0)
        chunk(_NCH - 1, 0, False)
        pltpu.make_async_copy(rows_v.at[0], out_hbm.at[pl.ds(0, _CH)],
                              wsem0).wait()
        pltpu.make_async_copy(rows_v.at[1], out_hbm.at[pl.ds(0, _CH)],
                              wsem1).wait()

    return k


def _sc_gather(table, idx4, slab):
    return _mk_sc_gather(slab)(table, idx4)


@functools.lru_cache(maxsize=None)
def _mk_sc_segsum(slab):
    @functools.partial(
        pl.kernel,
        out_type=jax.ShapeDtypeStruct((_NC, _N, _H), jnp.float32),
        mesh=_sc_mesh(),
        scratch_types=[
            pltpu.VMEM((_NSUB, _GB), jnp.int32),
            pltpu.VMEM((2, _GB, _H), jnp.float32),
            pltpu.VMEM_SHARED((_N, _H), jnp.float32),
            pltpu.SemaphoreType.DMA,
            pltpu.SemaphoreType.DMA,
        ],
    )
    def k(val_hbm, idx_hbm, zero_hbm, out_hbm, idx_v, val_v, acc,
          lsem0, lsem1):
        c = lax.axis_index("c")
        s = lax.axis_index("s")
        r0 = pl.multiple_of(s * _RPT, 8)
        rlo = (_NS - 1) * _RPT

        @pl.when(s < _NS - 1)
        def _zero_main():
            pltpu.sync_copy(zero_hbm.at[pl.ds(r0, _RPT)],
                            acc.at[pl.ds(r0, _RPT)])

        @pl.when(s == _NS - 1)
        def _zero_last():
            pltpu.sync_copy(zero_hbm.at[pl.ds(rlo, _RLAST)],
                            acc.at[pl.ds(rlo, _RLAST)])

        plsc.subcore_barrier()

        wid = lax.axis_index("s") * _NC + lax.axis_index("c")
        base = wid * _EPW
        pltpu.sync_copy(idx_hbm.at[slab, wid], idx_v)

        def load(i, b):
            off = pl.multiple_of(base + i * _GB, 8)
            pltpu.async_copy(val_hbm.at[pl.ds(off, _GB)], val_v.at[b],
                             lsem0 if b == 0 else lsem1)

        def drain(b):
            pltpu.make_async_copy(val_hbm.at[pl.ds(0, _GB)], val_v.at[b],
                                  lsem0 if b == 0 else lsem1).wait()

        def scat(i, b):
            pltpu.sync_copy(val_v.at[b], acc.at[idx_v.at[i]], add=True)

        load(0, 0)

        def body(i2, carry):
            i0 = i2 * 2
            drain(0)
            load(i0 + 1, 1)
            scat(i0, 0)
            drain(1)
            load(i0 + 2, 0)
            scat(i0 + 1, 1)
            return carry

        lax.fori_loop(0, (_NSUB - 1) // 2, body, 0)
        drain(0)
        scat(_NSUB - 1, 0)

        plsc.subcore_barrier()

        @pl.when(s < _NS - 1)
        def _out_main():
            pltpu.sync_copy(acc.at[pl.ds(r0, _RPT)],
                            out_hbm.at[c, pl.ds(r0, _RPT)])

        @pl.when(s == _NS - 1)
        def _out_last():
            pltpu.sync_copy(acc.at[pl.ds(rlo, _RLAST)],
                            out_hbm.at[c, pl.ds(rlo, _RLAST)])

    return k


def _sc_segsum(values, idx4, zeros, slab):
    return _mk_sc_segsum(slab)(values, idx4, zeros)


def _full(shape):
    return pl.BlockSpec(shape, lambda *_: tuple(0 for _ in shape))


def _tc_edge_pre(edge_attr_t, We):
    def body(ea_ref, w_ref, out_ref):
        out_ref[...] = lax.dot_general(ea_ref[...], w_ref[...],
                                       (((0,), (0,)), ((), ())),
                                       preferred_element_type=jnp.float32)

    ce = 2560
    return pl.pallas_call(
        body,
        grid=(_E // ce,),
        in_specs=[
            pl.BlockSpec((_DE, ce), lambda i: (0, i)),
            pl.BlockSpec((_DE, _H), lambda i: (0, 0)),
        ],
        out_specs=pl.BlockSpec((ce, _H), lambda i: (i, 0)),
        out_shape=jax.ShapeDtypeStruct((_E, _H), jnp.float32),
    )(edge_attr_t, We)


def _tc_node_pre(x, Wx, b):
    def body(x_ref, w_ref, b_ref, out_ref):
        out_ref[...] = jnp.dot(x_ref[...], w_ref[...],
                               preferred_element_type=jnp.float32) + b_ref[...]

    return pl.pallas_call(
        body,
        in_specs=[_full((_N, _D)), _full((_D, _H)), _full((1, _H))],
        out_specs=_full((_N, _H)),
        out_shape=jax.ShapeDtypeStruct((_N, _H), jnp.float32),
    )(x, Wx, b)


def _tc_init_combine(gxa, ea, slab):
    def body(g_ref, e_ref, h0_ref):
        h0_ref[...] = jnp.maximum(g_ref[...] + e_ref[...], 0.0)

    blk = pl.BlockSpec((_CE, _H), lambda i: (i, 0))
    off = slab * _GE
    eblk = pl.BlockSpec((_CE, _H), lambda i: (i + off, 0))
    return pl.pallas_call(
        body,
        grid=(_GE,),
        in_specs=[blk, eblk],
        out_specs=blk,
        out_shape=jax.ShapeDtypeStruct((_ES, _H), jnp.float32),
    )(gxa, ea)


def _tc_combine(p0, p1):
    def body(a_ref, b_ref, out_ref):
        out_ref[...] = (a_ref[0] + a_ref[1]) + (b_ref[0] + b_ref[1])

    return pl.pallas_call(
        body,
        in_specs=[_full((_NC, _N, _H)), _full((_NC, _N, _H))],
        out_specs=_full((_N, _H)),
        out_shape=jax.ShapeDtypeStruct((_N, _H), jnp.float32),
    )(p0, p1)


def _tc_step(gs, h, h0, Wh1, bh1, Wh2, bh2):
    def body(gs_ref, h_ref, h0_ref, w1_ref, b1_ref, w2_ref, b2_ref, out_ref):
        m = gs_ref[...] - h_ref[...]
        u = jnp.maximum(
            jnp.dot(m, w1_ref[...], preferred_element_type=jnp.float32)
            + b1_ref[...], 0.0)
        hid = jnp.dot(u, w2_ref[...], preferred_element_type=jnp.float32)
        out_ref[...] = jnp.maximum(h0_ref[...] + hid + b2_ref[...], 0.0)

    blk = pl.BlockSpec((_CE, _H), lambda i: (i, 0))
    wblk = pl.BlockSpec((_H, _H), lambda i: (0, 0))
    bblk = pl.BlockSpec((1, _H), lambda i: (0, 0))
    return pl.pallas_call(
        body, grid=(_GE,),
        in_specs=[blk, blk, blk, wblk, bblk, wblk, bblk],
        out_specs=blk,
        out_shape=jax.ShapeDtypeStruct((_ES, _H), jnp.float32),
    )(gs, h, h0, Wh1, bh1, Wh2, bh2)


def _tc_final(x, q0, q1, batch2d, Wf1, Wf2, bf, gamma, beta, Wq):
    def body(x_ref, m_ref, m2_ref, bat_ref, wf1_ref, wf2_ref, bf_ref, g_ref,
             be_ref, wq_ref, out_ref):
        m_node = (m_ref[0] + m_ref[1]) + (m2_ref[0] + m2_ref[1])
        h = jnp.dot(x_ref[...], wf1_ref[...], preferred_element_type=jnp.float32)
        h += jnp.dot(m_node, wf2_ref[...], preferred_element_type=jnp.float32)
        h = jnp.maximum(h + bf_ref[...], 0.0)
        mean = jnp.mean(h, axis=-1, keepdims=True)
        var = jnp.mean((h - mean) ** 2, axis=-1, keepdims=True)
        hn = (h - mean) * lax.rsqrt(var + 1e-5) * g_ref[...] + be_ref[...]
        hn = jnp.maximum(hn, 0.0)
        onehot = (bat_ref[...] ==
                  lax.broadcasted_iota(jnp.int32, (_N, _G), 1)).astype(jnp.float32)
        psum = lax.dot_general(onehot, hn, (((0,), (0,)), ((), ())),
                               preferred_element_type=jnp.float32)
        counts = lax.dot_general(onehot, jnp.ones((_N, 1), jnp.float32),
                                 (((0,), (0,)), ((), ())),
                                 preferred_element_type=jnp.float32)
        pooled = psum / jnp.maximum(counts, 1.0)
        out_ref[...] = jnp.dot(pooled, wq_ref[...],
                               preferred_element_type=jnp.float32)

    return pl.pallas_call(
        body,
        in_specs=[
            _full((_N, _D)), _full((_NC, _N, _H)), _full((_NC, _N, _H)),
            _full((_N, 1)),
            _full((_D, _H)), _full((_H, _H)), _full((1, _H)),
            _full((1, _H)), _full((1, _H)), _full((_H, _Q)),
        ],
        out_specs=_full((_G, _Q)),
        out_shape=jax.ShapeDtypeStruct((_G, _Q), jnp.float32),
    )(x, q0, q1, batch2d, Wf1, Wf2, bf, gamma, beta, Wq)


def kernel(x, edge_attr, edge_index, batch, W_init, b_init, Wh1, bh1, Wh2, bh2,
           W_fin, b_fin, gamma, beta, Wq):
    src4 = edge_index[0].reshape(_NSLAB, _NW, _NSUB, _GB)
    dst4 = edge_index[1].reshape(_NSLAB, _NW, _NSUB, _GB)
    ea_t = edge_attr.T
    Wx = W_init[:_D]
    We = W_init[_D:]
    Wf1 = W_fin[:_D]
    Wf2 = W_fin[_D:]
    b_init2 = b_init.reshape(1, _H)
    bh1_2 = bh1.reshape(1, _H)
    bh2_2 = bh2.reshape(1, _H)
    bf2 = b_fin.reshape(1, _H)
    gamma2 = gamma.reshape(1, _H)
    beta2 = beta.reshape(1, _H)
    zeros_nh = jnp.zeros((_N, _H), jnp.float32)

    XA = _tc_node_pre(x, Wx, b_init2)
    EA = _tc_edge_pre(ea_t, We)
    h0s = []
    for k in range(_NSLAB):
        gXA = _sc_gather(XA, src4, k)
        h0s.append(_tc_init_combine(gXA, EA, k))

    hs = list(h0s)
    for _ in range(_STEPS):
        parts = [_sc_segsum(hs[k], dst4, zeros_nh, k) for k in range(_NSLAB)]
        S = _tc_combine(parts[0], parts[1])
        gSs = [_sc_gather(S, dst4, k) for k in range(_NSLAB)]
        hs = [_tc_step(gSs[k], hs[k], h0s[k], Wh1, bh1_2, Wh2, bh2_2)
              for k in range(_NSLAB)]

    q = [_sc_segsum(hs[k], src4, zeros_nh, k) for k in range(_NSLAB)]
    return _tc_final(x, q[0], q[1], batch.reshape(_N, 1), Wf1, Wf2, bf2,
                     gamma2, beta2, Wq)


# speedup vs baseline: 3.1468x; 1.0722x over previous
import functools

import jax
import jax.numpy as jnp
from jax import lax
from jax.experimental import pallas as pl
from jax.experimental.pallas import tpu as pltpu
from jax.experimental.pallas import tpu_sc as plsc

_N = 10000
_E = 320000
_D = 128
_DE = 16
_H = 128
_Q = 16
_G = 64
_STEPS = 2

_NC = 2
_NS = 16
_NW = _NC * _NS
_NSLAB = 2
_ES = _E // _NSLAB
_EPW = _ES // _NW
_GB = 40
_NSUB = _EPW // _GB
_GSUB = 5
_CH = _GB * _GSUB
_NCH = _EPW // _CH
_RPT = 624
_RLAST = _N - (_NS - 1) * _RPT

_CE = 2000
_GE = _ES // _CE


def _sc_mesh():
    return plsc.VectorSubcoreMesh(core_axis_name="c", subcore_axis_name="s")


@functools.lru_cache(maxsize=None)
def _mk_sc_gather(slab):
    @functools.partial(
        pl.kernel,
        out_type=jax.ShapeDtypeStruct((_ES, _H), jnp.float32),
        mesh=_sc_mesh(),
        scratch_types=[
            pltpu.VMEM((_NSUB, _GB), jnp.int32),
            pltpu.VMEM((2, _CH, _H), jnp.float32),
            pltpu.SemaphoreType.DMA,
            pltpu.SemaphoreType.DMA,
            pltpu.SemaphoreType.DMA,
        ],
    )
    def k(table_hbm, idx_hbm, out_hbm, idx_v, rows_v, gsem, wsem0, wsem1):
        wid = lax.axis_index("s") * _NC + lax.axis_index("c")
        base = wid * _EPW
        pltpu.sync_copy(idx_hbm.at[slab, wid], idx_v)

        def chunk(i, b, first):
            if not first:
                pltpu.make_async_copy(
                    rows_v.at[b], out_hbm.at[pl.ds(0, _CH)],
                    wsem0 if b == 0 else wsem1).wait()
            ds = [pltpu.async_copy(table_hbm.at[idx_v.at[i * _GSUB + j]],
                                   rows_v.at[b, pl.ds(j * _GB, _GB)], gsem)
                  for j in range(_GSUB)]
            for d in ds:
                d.wait()
            off = pl.multiple_of(base + i * _CH, 8)
            pltpu.async_copy(rows_v.at[b], out_hbm.at[pl.ds(off, _CH)],
                             wsem0 if b == 0 else wsem1)

        chunk(0, 0, True)
        chunk(1, 1, True)

        def body(i2, carry):
            chunk(2 + i2 * 2, 0, False)
            chunk(3 + i2 * 2, 1, False)
            return carry

        lax.fori_loop(0, (_NCH - 3) // 2, body, 0)
        chunk(_NCH - 1, 0, False)
        pltpu.make_async_copy(rows_v.at[0], out_hbm.at[pl.ds(0, _CH)],
                              wsem0).wait()
        pltpu.make_async_copy(rows_v.at[1], out_hbm.at[pl.ds(0, _CH)],
                              wsem1).wait()

    return k


def _sc_gather(table, idx4, slab):
    return _mk_sc_gather(slab)(table, idx4)


@functools.lru_cache(maxsize=None)
def _mk_sc_segsum(slab):
    @functools.partial(
        pl.kernel,
        out_type=jax.ShapeDtypeStruct((_NC, _N, _H), jnp.float32),
        mesh=_sc_mesh(),
        scratch_types=[
            pltpu.VMEM((_NSUB, _GB), jnp.int32),
            pltpu.VMEM((2, _GB, _H), jnp.float32),
            pltpu.VMEM_SHARED((_N, _H), jnp.float32),
            pltpu.SemaphoreType.DMA,
            pltpu.SemaphoreType.DMA,
            pltpu.SemaphoreType.DMA,
            pltpu.SemaphoreType.DMA,
        ],
    )
    def k(val_hbm, idx_hbm, zero_hbm, out_hbm, idx_v, val_v, acc,
          lsem0, lsem1, ssem0, ssem1):
        c = lax.axis_index("c")
        s = lax.axis_index("s")
        r0 = pl.multiple_of(s * _RPT, 8)
        rlo = (_NS - 1) * _RPT

        @pl.when(s < _NS - 1)
        def _zero_main():
            pltpu.sync_copy(zero_hbm.at[pl.ds(r0, _RPT)],
                            acc.at[pl.ds(r0, _RPT)])

        @pl.when(s == _NS - 1)
        def _zero_last():
            pltpu.sync_copy(zero_hbm.at[pl.ds(rlo, _RLAST)],
                            acc.at[pl.ds(rlo, _RLAST)])

        plsc.subcore_barrier()

        wid = lax.axis_index("s") * _NC + lax.axis_index("c")
        base = wid * _EPW
        pltpu.sync_copy(idx_hbm.at[slab, wid], idx_v)

        def load(i, b):
            off = pl.multiple_of(base + i * _GB, 8)
            pltpu.async_copy(val_hbm.at[pl.ds(off, _GB)], val_v.at[b],
                             lsem0 if b == 0 else lsem1)

        def drain(b):
            pltpu.make_async_copy(val_hbm.at[pl.ds(0, _GB)], val_v.at[b],
                                  lsem0 if b == 0 else lsem1).wait()

        def scat(i, b):
            return pltpu.async_copy(val_v.at[b], acc.at[idx_v.at[i]],
                                    ssem0 if b == 0 else ssem1, add=True)

        load(0, 0)
        load(1, 1)

        def body(i2, carry):
            i0 = i2 * 2
            drain(0)
            d0 = scat(i0, 0)
            drain(1)
            d1 = scat(i0 + 1, 1)
            d0.wait()
            load(i0 + 2, 0)
            d1.wait()
            load(i0 + 3, 1)
            return carry

        lax.fori_loop(0, (_NSUB - 3) // 2, body, 0)
        drain(0)
        d0 = scat(_NSUB - 3, 0)
        drain(1)
        d1 = scat(_NSUB - 2, 1)
        d0.wait()
        load(_NSUB - 1, 0)
        d1.wait()
        drain(0)
        scat(_NSUB - 1, 0).wait()

        plsc.subcore_barrier()

        @pl.when(s < _NS - 1)
        def _out_main():
            pltpu.sync_copy(acc.at[pl.ds(r0, _RPT)],
                            out_hbm.at[c, pl.ds(r0, _RPT)])

        @pl.when(s == _NS - 1)
        def _out_last():
            pltpu.sync_copy(acc.at[pl.ds(rlo, _RLAST)],
                            out_hbm.at[c, pl.ds(rlo, _RLAST)])

    return k


def _sc_segsum(values, idx4, zeros, slab):
    return _mk_sc_segsum(slab)(values, idx4, zeros)


def _full(shape):
    return pl.BlockSpec(shape, lambda *_: tuple(0 for _ in shape))


def _tc_edge_pre(edge_attr_t, We, slab):
    def body(ea_ref, w_ref, out_ref):
        out_ref[...] = lax.dot_general(ea_ref[...], w_ref[...],
                                       (((0,), (0,)), ((), ())),
                                       preferred_element_type=jnp.float32)

    ce = 1280
    off = slab * (_ES // ce)
    return pl.pallas_call(
        body,
        grid=(_ES // ce,),
        in_specs=[
            pl.BlockSpec((_DE, ce), lambda i: (0, i + off)),
            pl.BlockSpec((_DE, _H), lambda i: (0, 0)),
        ],
        out_specs=pl.BlockSpec((ce, _H), lambda i: (i, 0)),
        out_shape=jax.ShapeDtypeStruct((_ES, _H), jnp.float32),
    )(edge_attr_t, We)


def _tc_node_pre(x, Wx, b):
    def body(x_ref, w_ref, b_ref, out_ref):
        out_ref[...] = jnp.dot(x_ref[...], w_ref[...],
                               preferred_element_type=jnp.float32) + b_ref[...]

    return pl.pallas_call(
        body,
        in_specs=[_full((_N, _D)), _full((_D, _H)), _full((1, _H))],
        out_specs=_full((_N, _H)),
        out_shape=jax.ShapeDtypeStruct((_N, _H), jnp.float32),
    )(x, Wx, b)


def _tc_init_combine(gxa, ea, slab):
    def body(g_ref, e_ref, h0_ref):
        h0_ref[...] = jnp.maximum(g_ref[...] + e_ref[...], 0.0)

    blk = pl.BlockSpec((_CE, _H), lambda i: (i, 0))
    return pl.pallas_call(
        body,
        grid=(_GE,),
        in_specs=[blk, blk],
        out_specs=blk,
        out_shape=jax.ShapeDtypeStruct((_ES, _H), jnp.float32),
    )(gxa, ea)


def _tc_combine(p0, p1):
    def body(a_ref, b_ref, out_ref):
        out_ref[...] = (a_ref[0] + a_ref[1]) + (b_ref[0] + b_ref[1])

    return pl.pallas_call(
        body,
        in_specs=[_full((_NC, _N, _H)), _full((_NC, _N, _H))],
        out_specs=_full((_N, _H)),
        out_shape=jax.ShapeDtypeStruct((_N, _H), jnp.float32),
    )(p0, p1)


def _tc_step(gs, h, h0, Wh1, bh1, Wh2, bh2):
    def body(gs_ref, h_ref, h0_ref, w1_ref, b1_ref, w2_ref, b2_ref, out_ref):
        m = gs_ref[...] - h_ref[...]
        u = jnp.maximum(
            jnp.dot(m, w1_ref[...], preferred_element_type=jnp.float32)
            + b1_ref[...], 0.0)
        hid = jnp.dot(u, w2_ref[...], preferred_element_type=jnp.float32)
        out_ref[...] = jnp.maximum(h0_ref[...] + hid + b2_ref[...], 0.0)

    blk = pl.BlockSpec((_CE, _H), lambda i: (i, 0))
    wblk = pl.BlockSpec((_H, _H), lambda i: (0, 0))
    bblk = pl.BlockSpec((1, _H), lambda i: (0, 0))
    return pl.pallas_call(
        body, grid=(_GE,),
        in_specs=[blk, blk, blk, wblk, bblk, wblk, bblk],
        out_specs=blk,
        out_shape=jax.ShapeDtypeStruct((_ES, _H), jnp.float32),
    )(gs, h, h0, Wh1, bh1, Wh2, bh2)


def _tc_final(x, q0, q1, batch2d, Wf1, Wf2, bf, gamma, beta, Wq):
    def body(x_ref, m_ref, m2_ref, bat_ref, wf1_ref, wf2_ref, bf_ref, g_ref,
             be_ref, wq_ref, out_ref):
        m_node = (m_ref[0] + m_ref[1]) + (m2_ref[0] + m2_ref[1])
        h = jnp.dot(x_ref[...], wf1_ref[...], preferred_element_type=jnp.float32)
        h += jnp.dot(m_node, wf2_ref[...], preferred_element_type=jnp.float32)
        h = jnp.maximum(h + bf_ref[...], 0.0)
        mean = jnp.mean(h, axis=-1, keepdims=True)
        var = jnp.mean((h - mean) ** 2, axis=-1, keepdims=True)
        hn = (h - mean) * lax.rsqrt(var + 1e-5) * g_ref[...] + be_ref[...]
        hn = jnp.maximum(hn, 0.0)
        onehot = (bat_ref[...] ==
                  lax.broadcasted_iota(jnp.int32, (_N, _G), 1)).astype(jnp.float32)
        psum = lax.dot_general(onehot, hn, (((0,), (0,)), ((), ())),
                               preferred_element_type=jnp.float32)
        counts = lax.dot_general(onehot, jnp.ones((_N, 1), jnp.float32),
                                 (((0,), (0,)), ((), ())),
                                 preferred_element_type=jnp.float32)
        pooled = psum / jnp.maximum(counts, 1.0)
        out_ref[...] = jnp.dot(pooled, wq_ref[...],
                               preferred_element_type=jnp.float32)

    return pl.pallas_call(
        body,
        in_specs=[
            _full((_N, _D)), _full((_NC, _N, _H)), _full((_NC, _N, _H)),
            _full((_N, 1)),
            _full((_D, _H)), _full((_H, _H)), _full((1, _H)),
            _full((1, _H)), _full((1, _H)), _full((_H, _Q)),
        ],
        out_specs=_full((_G, _Q)),
        out_shape=jax.ShapeDtypeStruct((_G, _Q), jnp.float32),
    )(x, q0, q1, batch2d, Wf1, Wf2, bf, gamma, beta, Wq)


def kernel(x, edge_attr, edge_index, batch, W_init, b_init, Wh1, bh1, Wh2, bh2,
           W_fin, b_fin, gamma, beta, Wq):
    src4 = edge_index[0].reshape(_NSLAB, _NW, _NSUB, _GB)
    dst4 = edge_index[1].reshape(_NSLAB, _NW, _NSUB, _GB)
    ea_t = edge_attr.T
    Wx = W_init[:_D]
    We = W_init[_D:]
    Wf1 = W_fin[:_D]
    Wf2 = W_fin[_D:]
    b_init2 = b_init.reshape(1, _H)
    bh1_2 = bh1.reshape(1, _H)
    bh2_2 = bh2.reshape(1, _H)
    bf2 = b_fin.reshape(1, _H)
    gamma2 = gamma.reshape(1, _H)
    beta2 = beta.reshape(1, _H)
    zeros_nh = jnp.zeros((_N, _H), jnp.float32)

    XA = _tc_node_pre(x, Wx, b_init2)
    h0s = []
    for k in range(_NSLAB):
        gXA = _sc_gather(XA, src4, k)
        EA = _tc_edge_pre(ea_t, We, k)
        h0s.append(_tc_init_combine(gXA, EA, k))

    hs = list(h0s)
    for _ in range(_STEPS):
        parts = [_sc_segsum(hs[k], dst4, zeros_nh, k) for k in range(_NSLAB)]
        S = _tc_combine(parts[0], parts[1])
        gSs = [_sc_gather(S, dst4, k) for k in range(_NSLAB)]
        hs = [_tc_step(gSs[k], hs[k], h0s[k], Wh1, bh1_2, Wh2, bh2_2)
              for k in range(_NSLAB)]

    q = [_sc_segsum(hs[k], src4, zeros_nh, k) for k in range(_NSLAB)]
    return _tc_final(x, q[0], q[1], batch.reshape(_N, 1), Wf1, Wf2, bf2,
                     gamma2, beta2, Wq)
